# baseline (device time: 40537 ns/iter reference)
import jax
import jax.numpy as jnp
from jax import lax
from jax.experimental import pallas as pl
from jax.experimental.pallas import tpu as pltpu


def kernel(x, dy):
    k, m = x.shape
    _, f = dy.shape
    fh = f // 2
    mh = m // 2

    NC = 16
    fc = fh // NC
    BLOCKS = [2, 4, 4, 6]
    NB = len(BLOCKS)
    STARTS = [sum(BLOCKS[:i]) for i in range(NB)]
    dims = (((1,), (0,)), ((), ()))

    def body(x_ref, dy_ref, out_ref, xv_ref, dyv_ref, ps_ref, pm_ref, rx_ref,
             ys_ref, yr_ref, xsem, dysems, sxs, rxs, sys_, rys):
        my_x = lax.axis_index("x")
        my_y = lax.axis_index("y")

        x_copy = pltpu.make_async_copy(x_ref, xv_ref, xsem)
        x_copy.start()
        dy_copies = []
        for b in range(NB):
            s, w = STARTS[b] * fc, BLOCKS[b] * fc
            cp = pltpu.make_async_copy(
                dy_ref.at[:, pl.ds(my_y * fh + s, w)],
                dyv_ref.at[:, pl.ds(s, w)],
                dysems.at[b],
            )
            cp.start()
            dy_copies.append(cp)

        barrier = pltpu.get_barrier_semaphore()
        pl.semaphore_signal(
            barrier, inc=1, device_id=(1 - my_x, my_y),
            device_id_type=pl.DeviceIdType.MESH,
        )
        pl.semaphore_signal(
            barrier, inc=1, device_id=(my_x, 1 - my_y),
            device_id_type=pl.DeviceIdType.MESH,
        )
        x_copy.wait()
        xo = xv_ref[:, pl.ds((1 - my_x) * mh, mh)].T
        xm = xv_ref[:, pl.ds(my_x * mh, mh)].T

        pl.semaphore_wait(barrier, 2)

        rdmas_x = []
        for b in range(NB):
            dy_copies[b].wait()
            s, w = STARTS[b] * fc, BLOCKS[b] * fc
            ps_ref[:, pl.ds(s, w)] = lax.dot_general(
                xo, dyv_ref[:, pl.ds(s, w)], dims,
                preferred_element_type=jnp.float32,
            ).astype(jnp.bfloat16)
            for i in range(BLOCKS[b]):
                c = STARTS[b] + i
                r = pltpu.make_async_remote_copy(
                    src_ref=ps_ref.at[:, pl.ds(c * fc, fc)],
                    dst_ref=rx_ref.at[:, pl.ds(c * fc, fc)],
                    send_sem=sxs.at[c],
                    recv_sem=rxs.at[c],
                    device_id=(1 - my_x, my_y),
                    device_id_type=pl.DeviceIdType.MESH,
                )
                r.start()
                rdmas_x.append(r)

        for b in range(NB):
            s, w = STARTS[b] * fc, BLOCKS[b] * fc
            pm_ref[:, pl.ds(s, w)] = lax.dot_general(
                xm, dyv_ref[:, pl.ds(s, w)], dims,
                preferred_element_type=jnp.float32,
            )

        rdmas_y = []
        for c in range(NC):
            rdmas_x[c].wait_recv()
            s = (
                pm_ref[:, pl.ds(c * fc, fc)]
                + rx_ref[:, pl.ds(c * fc, fc)].astype(jnp.float32)
            )
            out_ref[:, pl.ds(my_y * fh + c * fc, fc)] = s
            ys_ref[:, pl.ds(c * fc, fc)] = s.astype(jnp.bfloat16)
            r = pltpu.make_async_remote_copy(
                src_ref=ys_ref.at[:, pl.ds(c * fc, fc)],
                dst_ref=yr_ref.at[:, pl.ds(c * fc, fc)],
                send_sem=sys_.at[c],
                recv_sem=rys.at[c],
                device_id=(my_x, 1 - my_y),
                device_id_type=pl.DeviceIdType.MESH,
            )
            r.start()
            rdmas_y.append(r)

        for c in range(NC):
            rdmas_y[c].wait_recv()
            out_ref[:, pl.ds((1 - my_y) * fh + c * fc, fc)] = (
                yr_ref[:, pl.ds(c * fc, fc)].astype(jnp.float32)
            )
        for c in range(NC):
            rdmas_y[c].wait_send()
            rdmas_x[c].wait_send()

    return pl.pallas_call(
        body,
        out_shape=jax.ShapeDtypeStruct((mh, f), jnp.float32),
        in_specs=[
            pl.BlockSpec(memory_space=pltpu.HBM),
            pl.BlockSpec(memory_space=pltpu.HBM),
        ],
        out_specs=pl.BlockSpec(memory_space=pltpu.VMEM),
        scratch_shapes=[
            pltpu.VMEM((k, m), jnp.float32),
            pltpu.VMEM((k, fh), jnp.float32),
            pltpu.VMEM((mh, fh), jnp.bfloat16),
            pltpu.VMEM((mh, fh), jnp.float32),
            pltpu.VMEM((mh, fh), jnp.bfloat16),
            pltpu.VMEM((mh, fh), jnp.bfloat16),
            pltpu.VMEM((mh, fh), jnp.bfloat16),
            pltpu.SemaphoreType.DMA,
            pltpu.SemaphoreType.DMA((NB,)),
            pltpu.SemaphoreType.DMA((NC,)),
            pltpu.SemaphoreType.DMA((NC,)),
            pltpu.SemaphoreType.DMA((NC,)),
            pltpu.SemaphoreType.DMA((NC,)),
        ],
        compiler_params=pltpu.CompilerParams(collective_id=0),
    )(x, dy)


# device time: 40400 ns/iter; 1.0034x vs baseline; 1.0034x over previous
import jax
import jax.numpy as jnp
from jax import lax
from jax.experimental import pallas as pl
from jax.experimental.pallas import tpu as pltpu


def kernel(x, dy):
    k, m = x.shape
    _, f = dy.shape
    fh = f // 2
    mh = m // 2

    NC = 16
    fc = fh // NC
    NB = 4
    fb = fh // NB
    CPB = NC // NB
    dims = (((1,), (0,)), ((), ()))

    def body(x_ref, dy_ref, out_ref, xv_ref, dyv_ref, ps_ref, pm_ref, rx_ref,
             ys_ref, yr_ref, xsem, dysems, sxs, rxs, sys_, rys):
        my_x = lax.axis_index("x")
        my_y = lax.axis_index("y")

        x_copy = pltpu.make_async_copy(x_ref, xv_ref, xsem)
        x_copy.start()
        dy_copies = []
        for b in range(NB):
            cp = pltpu.make_async_copy(
                dy_ref.at[:, pl.ds(my_y * fh + b * fb, fb)],
                dyv_ref.at[:, pl.ds(b * fb, fb)],
                dysems.at[b],
            )
            cp.start()
            dy_copies.append(cp)

        barrier = pltpu.get_barrier_semaphore()
        pl.semaphore_signal(
            barrier, inc=1, device_id=(1 - my_x, my_y),
            device_id_type=pl.DeviceIdType.MESH,
        )
        pl.semaphore_signal(
            barrier, inc=1, device_id=(my_x, 1 - my_y),
            device_id_type=pl.DeviceIdType.MESH,
        )
        x_copy.wait()
        xo = xv_ref[:, pl.ds((1 - my_x) * mh, mh)].T
        xm = xv_ref[:, pl.ds(my_x * mh, mh)].T

        pl.semaphore_wait(barrier, 2)

        rdmas_x = []
        for b in range(NB):
            dy_copies[b].wait()
            dyb = dyv_ref[:, pl.ds(b * fb, fb)]
            ps_ref[:, pl.ds(b * fb, fb)] = lax.dot_general(
                xo, dyb, dims, preferred_element_type=jnp.float32
            ).astype(jnp.bfloat16)
            for i in range(CPB):
                c = b * CPB + i
                r = pltpu.make_async_remote_copy(
                    src_ref=ps_ref.at[:, pl.ds(c * fc, fc)],
                    dst_ref=rx_ref.at[:, pl.ds(c * fc, fc)],
                    send_sem=sxs.at[c],
                    recv_sem=rxs.at[c],
                    device_id=(1 - my_x, my_y),
                    device_id_type=pl.DeviceIdType.MESH,
                )
                r.start()
                rdmas_x.append(r)

        for b in range(NB):
            dyb = dyv_ref[:, pl.ds(b * fb, fb)]
            pm_ref[:, pl.ds(b * fb, fb)] = lax.dot_general(
                xm, dyb, dims, preferred_element_type=jnp.float32
            )

        rdmas_y = []
        for c in range(NC):
            rdmas_x[c].wait_recv()
            s = (
                pm_ref[:, pl.ds(c * fc, fc)]
                + rx_ref[:, pl.ds(c * fc, fc)].astype(jnp.float32)
            )
            out_ref[:, pl.ds(my_y * fh + c * fc, fc)] = s
            ys_ref[:, pl.ds(c * fc, fc)] = s.astype(jnp.bfloat16)
            r = pltpu.make_async_remote_copy(
                src_ref=ys_ref.at[:, pl.ds(c * fc, fc)],
                dst_ref=yr_ref.at[:, pl.ds(c * fc, fc)],
                send_sem=sys_.at[c],
                recv_sem=rys.at[c],
                device_id=(my_x, 1 - my_y),
                device_id_type=pl.DeviceIdType.MESH,
            )
            r.start()
            rdmas_y.append(r)

        for c in range(NC):
            rdmas_y[c].wait_recv()
            out_ref[:, pl.ds((1 - my_y) * fh + c * fc, fc)] = (
                yr_ref[:, pl.ds(c * fc, fc)].astype(jnp.float32)
            )
        for c in range(NC):
            rdmas_y[c].wait_send()
            rdmas_x[c].wait_send()

    return pl.pallas_call(
        body,
        out_shape=jax.ShapeDtypeStruct((mh, f), jnp.float32),
        in_specs=[
            pl.BlockSpec(memory_space=pltpu.HBM),
            pl.BlockSpec(memory_space=pltpu.HBM),
        ],
        out_specs=pl.BlockSpec(memory_space=pltpu.VMEM),
        scratch_shapes=[
            pltpu.VMEM((k, m), jnp.float32),
            pltpu.VMEM((k, fh), jnp.float32),
            pltpu.VMEM((mh, fh), jnp.bfloat16),
            pltpu.VMEM((mh, fh), jnp.float32),
            pltpu.VMEM((mh, fh), jnp.bfloat16),
            pltpu.VMEM((mh, fh), jnp.bfloat16),
            pltpu.VMEM((mh, fh), jnp.bfloat16),
            pltpu.SemaphoreType.DMA,
            pltpu.SemaphoreType.DMA((NB,)),
            pltpu.SemaphoreType.DMA((NC,)),
            pltpu.SemaphoreType.DMA((NC,)),
            pltpu.SemaphoreType.DMA((NC,)),
            pltpu.SemaphoreType.DMA((NC,)),
        ],
        compiler_params=pltpu.CompilerParams(collective_id=0),
    )(x, dy)
